# baseline (speedup 1.0000x reference)
import jax
import jax.numpy as jnp
from jax.experimental import pallas as pl
from jax.experimental.pallas import tpu as pltpu

_K = 10
_T = 2.0
_BIG = 3.0e38
_GRID = 2


def _loss_kernel(f_ref, fblk_ref, labc_ref, labr_ref, out_ref):
    f = f_ref[:]
    fb = fblk_ref[:]
    lab_col = labc_ref[:]
    lab_row = labr_ref[:]
    n = f.shape[0]
    nb = fb.shape[0]
    m = n - 1
    inv_m = jnp.float32(1.0 / m)
    g = jax.lax.dot_general(f, fb, (((1,), (1,)), ((), ())),
                            preferred_element_type=jnp.float32)
    i = pl.program_id(0)
    row = jax.lax.broadcasted_iota(jnp.int32, (n, nb), 0)
    col = jax.lax.broadcasted_iota(jnp.int32, (n, nb), 1)
    eye = row == col + i * nb
    sqn_col = jnp.sum(f * f, axis=1, keepdims=True)
    sqnb = jnp.sum(fb * fb, axis=1, keepdims=True)
    sel = jnp.where(jax.lax.broadcasted_iota(jnp.int32, (nb, nb), 0) ==
                    jax.lax.broadcasted_iota(jnp.int32, (nb, nb), 1), 1.0, 0.0)
    sqn_row = jax.lax.dot_general(sqnb, sel, (((0,), (0,)), ((), ())),
                                  preferred_element_type=jnp.float32)
    sq = jnp.maximum(sqn_col + sqn_row - 2.0 * g, 0.0)
    l = jnp.sqrt(sq) * jnp.float32(1.0 / _T)
    l = jnp.where(eye, 0.0, l)

    d = jnp.abs(lab_col - lab_row)
    d = jnp.where(eye, _BIG, d)

    l_rows = jnp.sum(l, axis=1, keepdims=True)
    grand = jnp.sum(l_rows, axis=0, keepdims=True)

    c_run = jnp.zeros((1, nb), dtype=jnp.float32)
    sl_run = jnp.zeros((1, nb), dtype=jnp.float32)
    acc = jnp.zeros((1, nb), dtype=jnp.float32)
    prev = jnp.full((1, nb), -1.0, dtype=jnp.float32)
    for _ in range(_K):
        v = jnp.min(jnp.where(d > prev, d, _BIG), axis=0, keepdims=True)
        eq = d == v
        cnt = jnp.sum(jnp.where(eq, 1.0, 0.0), axis=0, keepdims=True)
        sl = jnp.sum(jnp.where(eq, l, 0.0), axis=0, keepdims=True)
        used = jnp.minimum(jnp.maximum(_K - c_run, 0.0), cnt)
        slot_l = sl * (used / jnp.maximum(cnt, 1.0))
        acc = acc + slot_l * (2.0 - 2.0 * c_run * inv_m)
        acc = acc + used * (2.0 * sl_run) * inv_m
        c_run = c_run + cnt
        sl_run = sl_run + sl
        prev = v
    total = jnp.sum(acc, axis=1, keepdims=True)
    total = total - jnp.float32(_K) * inv_m * grand
    out_ref[...] = (total * jnp.float32(1.0 / (_K * (_K - 1)))).reshape(1, 1, 1)


def kernel(features, labels, ranks):
    del ranks
    n = features.shape[0]
    nb = n // _GRID
    lab_col = labels.reshape(n, 1).astype(jnp.float32)
    lab_row = labels.reshape(1, n).astype(jnp.float32)
    out = pl.pallas_call(
        _loss_kernel,
        grid=(_GRID,),
        in_specs=[
            pl.BlockSpec((n, features.shape[1]), lambda i: (0, 0)),
            pl.BlockSpec((nb, features.shape[1]), lambda i: (i, 0)),
            pl.BlockSpec((n, 1), lambda i: (0, 0)),
            pl.BlockSpec((1, nb), lambda i: (0, i)),
        ],
        out_specs=pl.BlockSpec((1, 1, 1), lambda i: (i, 0, 0)),
        out_shape=jax.ShapeDtypeStruct((_GRID, 1, 1), jnp.float32),
        compiler_params=pltpu.CompilerParams(
            dimension_semantics=("parallel",)),
    )(features, features, lab_col, lab_row)
    return jnp.sum(out)


# speedup vs baseline: 1.2570x; 1.2570x over previous
import jax
import jax.numpy as jnp
from jax.experimental import pallas as pl
from jax.experimental.pallas import tpu as pltpu

_K = 10
_T = 2.0
_BIG = 3.0e38


def _loss_kernel(f_ref, labc_ref, labr_ref, out_ref):
    f = f_ref[:]
    n = f.shape[0]
    m = n - 1
    inv_m = jnp.float32(1.0 / m)
    g = jax.lax.dot_general(f, f, (((1,), (1,)), ((), ())),
                            preferred_element_type=jnp.float32)
    row = jax.lax.broadcasted_iota(jnp.int32, (n, n), 0)
    col = jax.lax.broadcasted_iota(jnp.int32, (n, n), 1)
    eye = row == col
    diag = jnp.where(eye, g, 0.0)
    sqn_col = jnp.sum(diag, axis=1, keepdims=True)
    sqn_row = jnp.sum(diag, axis=0, keepdims=True)
    sq = jnp.maximum(sqn_col + sqn_row - 2.0 * g, 0.0)
    l = jnp.sqrt(sq) * jnp.float32(1.0 / _T)
    l = jnp.where(eye, 0.0, l)

    d = jnp.abs(labc_ref[:] - labr_ref[:])
    d = jnp.where(eye, _BIG, d)

    t_l = jnp.sum(l, axis=0, keepdims=True)

    c_run = jnp.zeros((1, n), dtype=jnp.float32)
    sl_run = jnp.zeros((1, n), dtype=jnp.float32)
    prev = jnp.full((1, n), -1.0, dtype=jnp.float32)
    acc = jnp.zeros((1, n), dtype=jnp.float32)
    for _ in range(_K):
        v = jnp.min(jnp.where(d > prev, d, _BIG), axis=0, keepdims=True)
        eq = d == v
        cnt = jnp.sum(jnp.where(eq, 1.0, 0.0), axis=0, keepdims=True)
        sl = jnp.sum(jnp.where(eq, l, 0.0), axis=0, keepdims=True)
        used = jnp.minimum(jnp.maximum(_K - c_run, 0.0), cnt)
        slot_l = sl * (used / jnp.maximum(cnt, 1.0))
        acc = acc + slot_l * (2.0 - 2.0 * c_run * inv_m)
        acc = acc - used * (t_l - 2.0 * sl_run) * inv_m
        c_run = c_run + cnt
        sl_run = sl_run + sl
        prev = v
    total = jnp.sum(acc, axis=1, keepdims=True)
    out_ref[...] = total * jnp.float32(1.0 / (_K * (_K - 1)))


def kernel(features, labels, ranks):
    del ranks
    n = features.shape[0]
    lab_col = labels.reshape(n, 1).astype(jnp.float32)
    lab_row = labels.reshape(1, n).astype(jnp.float32)
    out = pl.pallas_call(
        _loss_kernel,
        out_shape=jax.ShapeDtypeStruct((1, 1), jnp.float32),
    )(features, lab_col, lab_row)
    return out[0, 0]
